# baseline (speedup 1.0000x reference)
import functools

import jax
import jax.numpy as jnp
from jax import lax
from jax.experimental import pallas as pl
from jax.experimental.pallas import tpu as pltpu

F32 = jnp.float32


def _phase_split_flat(x_nhwc):
    B, H, W, C = x_nhwc.shape
    xp = jnp.pad(x_nhwc, ((0, 0), (1, 1), (1, 1), (0, 0)))
    phases = [xp[:, a::2, b::2, :] for a in (0, 1) for b in (0, 1)]
    P = jnp.concatenate(phases, axis=-1)
    hp = H // 2 + 1
    P = P.reshape(B, hp * hp, 4 * C)
    return jnp.pad(P, ((0, 0), (0, 8), (0, 0)))

def _conv_tap_weights(conv_w):
    wt = jnp.transpose(conv_w, (2, 3, 1, 0))
    taps = []
    for s in (0, 1):
        for t in (0, 1):
            taps.append(jnp.concatenate(
                [wt[2 * s + a, 2 * t + b] for a in (0, 1) for b in (0, 1)],
                axis=0))
    return jnp.stack(taps, axis=0)

def _deconv_tap_weights(w):
    rows = []
    for py in (0, 1):
        cols = []
        for px in (0, 1):
            taps = []
            for s in (0, 1):
                dy = 3 - 2 * s if py == 0 else 2 - 2 * s
                for t in (0, 1):
                    dx = 3 - 2 * t if px == 0 else 2 - 2 * t
                    taps.append(w[:, :, dy, dx])
            cols.append(jnp.stack(taps, axis=0))
        rows.append(jnp.stack(cols, axis=0))
    return jnp.stack(rows, axis=0)

def _pad_flat(x_nhwc):
    B, h, w, C = x_nhwc.shape
    xp = jnp.pad(x_nhwc, ((0, 0), (1, 1), (1, 1), (0, 0)))
    xp = xp.reshape(B, (h + 2) * (w + 2), C)
    return jnp.pad(xp, ((0, 0), (0, 8), (0, 0)))


def _conv_body(p_ref, w_ref, b_ref, o_ref, *, hp, h, act):
    M = h * hp
    acc = jnp.zeros((M, w_ref.shape[2]), dtype=F32)
    for s in (0, 1):
        for t in (0, 1):
            base = s * hp + t
            a = p_ref[0, base:base + M, :]
            acc = acc + jnp.dot(a, w_ref[2 * s + t],
                                preferred_element_type=F32)
    acc = acc + b_ref[0]
    if act == "relu":
        acc = jnp.maximum(acc, 0.0)
    o_ref[0] = acc


def _conv_layer(P, Wt, b, *, B, hp, h, Cout, act):
    out = pl.pallas_call(
        functools.partial(_conv_body, hp=hp, h=h, act=act),
        grid=(B,),
        in_specs=[
            pl.BlockSpec((1, hp * hp + 8, Wt.shape[1]), lambda i: (i, 0, 0)),
            pl.BlockSpec(Wt.shape, lambda i: (0, 0, 0)),
            pl.BlockSpec((1, Cout), lambda i: (0, 0)),
        ],
        out_specs=pl.BlockSpec((1, h * hp, Cout), lambda i: (i, 0, 0)),
        out_shape=jax.ShapeDtypeStruct((B, h * hp, Cout), F32),
    )(P, Wt, b.reshape(1, -1))
    return out.reshape(B, h, hp, Cout)[:, :, :h, :]


def _deconv_body(x_ref, v_ref, b_ref, o_ref, *, wp, h, w, act):
    pid_y = pl.program_id(1)
    pid_x = pl.program_id(2)
    M = h * wp
    Cout = v_ref.shape[4]
    for py in (0, 1):
        for px in (0, 1):
            @pl.when(jnp.logical_and(pid_y == py, pid_x == px))
            def _(py=py, px=px):
                acc = jnp.zeros((M, Cout), dtype=F32)
                for s in (0, 1):
                    for t in (0, 1):
                        base = (py + s) * wp + (px + t)
                        a = x_ref[0, base:base + M, :]
                        acc = acc + jnp.dot(a, v_ref[0, 0, 2 * s + t],
                                            preferred_element_type=F32)
                acc = acc + b_ref[0]
                if act == "relu":
                    acc = jnp.maximum(acc, 0.0)
                elif act == "sigmoid":
                    acc = jax.nn.sigmoid(acc)
                o_ref[0, 0, 0] = acc


def _deconv_layer(x_nhwc, w, b, *, act):
    B, h, wdt, Cin = x_nhwc.shape
    Cout = w.shape[1]
    wp = wdt + 2
    Xf = _pad_flat(x_nhwc)
    V = _deconv_tap_weights(w)
    out = pl.pallas_call(
        functools.partial(_deconv_body, wp=wp, h=h, w=wdt, act=act),
        grid=(B, 2, 2),
        in_specs=[
            pl.BlockSpec((1, (h + 2) * wp + 8, Cin),
                         lambda i, py, px: (i, 0, 0)),
            pl.BlockSpec((1, 1, 4, Cin, Cout),
                         lambda i, py, px: (py, px, 0, 0, 0)),
            pl.BlockSpec((1, Cout), lambda i, py, px: (0, 0)),
        ],
        out_specs=pl.BlockSpec((1, 1, 1, h * wp, Cout),
                               lambda i, py, px: (i, py, px, 0, 0)),
        out_shape=jax.ShapeDtypeStruct((B, 2, 2, h * wp, Cout), F32),
    )(Xf, V, b.reshape(1, -1))
    out = out.reshape(B, 2, 2, h, wp, Cout)[:, :, :, :, :wdt, :]
    out = jnp.transpose(out, (0, 3, 1, 4, 2, 5))
    return out.reshape(B, 2 * h, 2 * wdt, Cout)


def _vq_body(f_ref, f2_ref, e_ref, e2_ref, q_ref, loss_ref, *, n_embed,
             scale):
    i = pl.program_id(0)
    f = f_ref[...]
    e = e_ref[...]
    scores = (f2_ref[...] + e2_ref[...]) - 2.0 * jnp.dot(f, e.T)
    smin = jnp.min(scores, axis=1, keepdims=True)
    iota = lax.broadcasted_iota(jnp.int32, scores.shape, 1)
    idx = jnp.min(jnp.where(scores <= smin, iota, n_embed), axis=1,
                  keepdims=True)
    onehot = (iota == idx).astype(F32)
    q = jnp.dot(onehot, e, preferred_element_type=F32,
                precision=lax.Precision.HIGHEST)
    q_ref[...] = q
    d = q - f
    part = jnp.sum(d * d, axis=(0, 1), keepdims=True) * scale

    @pl.when(i == 0)
    def _():
        loss_ref[...] = jnp.zeros((1, 1), F32)

    loss_ref[...] += part


def _vq(flat, embeddings, *, chunk):
    M, D = flat.shape
    N = embeddings.shape[0]
    n_chunks = M // chunk
    scale = (1.0 + 0.25) / (M * D)
    f2 = jnp.sum(flat ** 2, axis=1, keepdims=True)
    e2 = jnp.sum(embeddings ** 2, axis=1).reshape(1, N)
    q, loss = pl.pallas_call(
        functools.partial(_vq_body, n_embed=N, scale=scale),
        grid=(n_chunks,),
        in_specs=[
            pl.BlockSpec((chunk, D), lambda i: (i, 0)),
            pl.BlockSpec((chunk, 1), lambda i: (i, 0)),
            pl.BlockSpec((N, D), lambda i: (0, 0)),
            pl.BlockSpec((1, N), lambda i: (0, 0)),
        ],
        out_specs=[
            pl.BlockSpec((chunk, D), lambda i: (i, 0)),
            pl.BlockSpec((1, 1), lambda i: (0, 0)),
        ],
        out_shape=[
            jax.ShapeDtypeStruct((M, D), F32),
            jax.ShapeDtypeStruct((1, 1), F32),
        ],
    )(flat, f2, embeddings, e2)
    return q, loss[0, 0]


def kernel(x, conv1_w, conv1_b, conv2_w, conv2_b, conv3_w, conv3_b,
           embeddings, deconv1_w, deconv1_b, deconv2_w, deconv2_b,
           deconv3_w, deconv3_b):
    B = x.shape[0]
    h1 = x.shape[2] // 2
    z = _conv_layer(_phase_split_flat(jnp.transpose(x, (0, 2, 3, 1))),
                    _conv_tap_weights(conv1_w), conv1_b,
                    B=B, hp=h1 + 1, h=h1, Cout=conv1_w.shape[0], act="relu")
    h2 = h1 // 2
    z = _conv_layer(_phase_split_flat(z),
                    _conv_tap_weights(conv2_w), conv2_b,
                    B=B, hp=h2 + 1, h=h2, Cout=conv2_w.shape[0], act="relu")
    h3 = h2 // 2
    z = _conv_layer(_phase_split_flat(z),
                    _conv_tap_weights(conv3_w), conv3_b,
                    B=B, hp=h3 + 1, h=h3, Cout=conv3_w.shape[0], act="none")
    D = embeddings.shape[1]
    flat = z.reshape(B * h3 * h3, D)
    q, vq_loss = _vq(flat, embeddings, chunk=(B * h3 * h3) // 4)
    zq = q.reshape(B, h3, h3, D)
    h = _deconv_layer(zq, deconv1_w, deconv1_b, act="relu")
    h = _deconv_layer(h, deconv2_w, deconv2_b, act="relu")
    h = _deconv_layer(h, deconv3_w, deconv3_b, act="sigmoid")
    x_rec = jnp.transpose(h, (0, 3, 1, 2))
    return (x_rec, vq_loss.reshape(()))


# speedup vs baseline: 2.9918x; 2.9918x over previous
import functools

import jax
import jax.numpy as jnp
from jax import lax
from jax.experimental import pallas as pl

F32 = jnp.float32


def _phase_split(x_nhwc):
    xp = jnp.pad(x_nhwc, ((0, 0), (1, 1), (1, 1), (0, 0)))
    phases = [xp[:, a::2, b::2, :] for a in (0, 1) for b in (0, 1)]
    return jnp.concatenate(phases, axis=-1)


def _conv1_tap_weights(conv_w):
    wt = jnp.transpose(conv_w, (2, 3, 1, 0))
    taps = []
    for s in (0, 1):
        for t in (0, 1):
            taps.append(jnp.concatenate(
                [wt[2 * s + a, 2 * t + b] for a in (0, 1) for b in (0, 1)],
                axis=0))
    return jnp.stack(taps, axis=0)


def _conv_uv_weights(conv_w):
    wt = jnp.transpose(conv_w, (2, 3, 1, 0))
    return wt.reshape(16, wt.shape[2], wt.shape[3])


def _deconv_tap_weights(w):
    rows = []
    for py in (0, 1):
        cols = []
        for px in (0, 1):
            taps = []
            for s in (0, 1):
                dy = 3 - 2 * s if py == 0 else 2 - 2 * s
                for t in (0, 1):
                    dx = 3 - 2 * t if px == 0 else 2 - 2 * t
                    taps.append(w[:, :, dy, dx])
            cols.append(jnp.stack(taps, axis=0))
        rows.append(jnp.stack(cols, axis=0))
    return jnp.stack(rows, axis=0)


def _deconv3_stacked_weights(w):
    V = _deconv_tap_weights(w)
    Cin = w.shape[0]
    out = []
    for r in (0, 1, 2):
        for c in (0, 1, 2):
            blocks = []
            for py in (0, 1):
                for px in (0, 1):
                    s, t = r - py, c - px
                    if 0 <= s <= 1 and 0 <= t <= 1:
                        blocks.append(V[py, px, 2 * s + t])
                    else:
                        blocks.append(jnp.zeros((Cin, 3), F32))
            out.append(jnp.concatenate(blocks, axis=1))
    return jnp.stack(out, axis=0)


def _ring_pad(y):
    h, w, C = y.shape
    colz = jnp.zeros((h, 1, C), F32)
    y = jnp.concatenate([colz, y, colz], axis=1)
    rowz = jnp.zeros((1, w + 2, C), F32)
    return jnp.concatenate([rowz, y, rowz], axis=0)


def _conv1_body(p_ref, w_ref, b_ref, o_ref, *, h):
    acc = jnp.zeros((h * h, w_ref.shape[2]), dtype=F32)
    for s in (0, 1):
        for t in (0, 1):
            a = p_ref[0, s:s + h, t:t + h, :].reshape(h * h, w_ref.shape[1])
            acc = acc + jnp.dot(a, w_ref[2 * s + t],
                                preferred_element_type=F32)
    acc = jnp.maximum(acc + b_ref[0], 0.0)
    o_ref[0] = _ring_pad(acc.reshape(h, h, w_ref.shape[2]))


def _conv_body(x_ref, w_ref, b_ref, o_ref, *, h, act, pad_out):
    C = w_ref.shape[1]
    O = w_ref.shape[2]
    acc = jnp.zeros((h * h, O), dtype=F32)
    for u in range(4):
        for v in range(4):
            a = x_ref[0, u:u + 2 * h - 1:2, v:v + 2 * h - 1:2, :]
            acc = acc + jnp.dot(a.reshape(h * h, C), w_ref[4 * u + v],
                                preferred_element_type=F32)
    acc = acc + b_ref[0]
    if act == "relu":
        acc = jnp.maximum(acc, 0.0)
    if pad_out:
        o_ref[0] = _ring_pad(acc.reshape(h, h, O))
    else:
        o_ref[0] = acc


def _vq_body(f_ref, f2_ref, e_ref, e2_ref, q_ref, loss_ref, *, n_embed,
             scale, h):
    i = pl.program_id(0)
    f = f_ref[...]
    e = e_ref[...]
    scores = (f2_ref[...] + e2_ref[...]) - 2.0 * jnp.dot(f, e.T)
    smin = jnp.min(scores, axis=1, keepdims=True)
    iota = lax.broadcasted_iota(jnp.int32, scores.shape, 1)
    idx = jnp.min(jnp.where(scores <= smin, iota, n_embed), axis=1,
                  keepdims=True)
    onehot = (iota == idx).astype(F32)
    q = jnp.dot(onehot, e, preferred_element_type=F32,
                precision=lax.Precision.HIGHEST)
    q_ref[0] = _ring_pad(q.reshape(h, h, e.shape[1]))
    d = q - f
    part = jnp.sum(d * d, axis=(0, 1), keepdims=True) * scale

    @pl.when(i == 0)
    def _():
        loss_ref[...] = jnp.zeros((1, 1), F32)

    loss_ref[...] += part


def _deconv_body(x_ref, v_ref, b_ref, o_ref, *, h, act):
    pid_y = pl.program_id(1)
    pid_x = pl.program_id(2)
    O = v_ref.shape[4]

    @pl.when(jnp.logical_and(pid_y == 0, pid_x == 0))
    def _():
        o_ref[0] = jnp.zeros(o_ref.shape[1:], F32)

    for py in (0, 1):
        for px in (0, 1):
            @pl.when(jnp.logical_and(pid_y == py, pid_x == px))
            def _(py=py, px=px):
                acc = jnp.zeros((h * h, O), dtype=F32)
                for s in (0, 1):
                    for t in (0, 1):
                        a = x_ref[0, py + s:py + s + h, px + t:px + t + h, :]
                        acc = acc + jnp.dot(a.reshape(h * h, v_ref.shape[3]),
                                            v_ref[0, 0, 2 * s + t],
                                            preferred_element_type=F32)
                acc = acc + b_ref[0]
                if act == "relu":
                    acc = jnp.maximum(acc, 0.0)
                o_ref[0, 1 + py:1 + py + 2 * h:2, 1 + px:1 + px + 2 * h:2, :] = (
                    acc.reshape(h, h, O))


def _deconv3_body(x_ref, v_ref, b_ref, o_ref, *, h):
    Cin = v_ref.shape[1]
    acc = jnp.zeros((h * h, 12), dtype=F32)
    for r in (0, 1, 2):
        for c in (0, 1, 2):
            a = x_ref[0, r:r + h, c:c + h, :]
            acc = acc + jnp.dot(a.reshape(h * h, Cin), v_ref[3 * r + c],
                                preferred_element_type=F32)
    o_ref[0] = jax.nn.sigmoid(acc + b_ref[0])


def _conv1(P, Wt, b, *, B, h, Cout):
    return pl.pallas_call(
        functools.partial(_conv1_body, h=h),
        grid=(B,),
        in_specs=[
            pl.BlockSpec((1, h + 1, h + 1, Wt.shape[1]),
                         lambda i: (i, 0, 0, 0)),
            pl.BlockSpec(Wt.shape, lambda i: (0, 0, 0)),
            pl.BlockSpec((1, Cout), lambda i: (0, 0)),
        ],
        out_specs=pl.BlockSpec((1, h + 2, h + 2, Cout),
                               lambda i: (i, 0, 0, 0)),
        out_shape=jax.ShapeDtypeStruct((B, h + 2, h + 2, Cout), F32),
    )(P, Wt, b.reshape(1, -1))


def _conv(x, w, b, *, act, pad_out):
    B, hin2, _, C = x.shape
    h = (hin2 - 2) // 2
    O = w.shape[0]
    W16 = _conv_uv_weights(w)
    out_shape = ((B, h + 2, h + 2, O) if pad_out else (B, h * h, O))
    blk = ((1, h + 2, h + 2, O) if pad_out else (1, h * h, O))
    idx = ((lambda i: (i, 0, 0, 0)) if pad_out else (lambda i: (i, 0, 0)))
    return pl.pallas_call(
        functools.partial(_conv_body, h=h, act=act, pad_out=pad_out),
        grid=(B,),
        in_specs=[
            pl.BlockSpec((1, hin2, hin2, C), lambda i: (i, 0, 0, 0)),
            pl.BlockSpec(W16.shape, lambda i: (0, 0, 0)),
            pl.BlockSpec((1, O), lambda i: (0, 0)),
        ],
        out_specs=pl.BlockSpec(blk, idx),
        out_shape=jax.ShapeDtypeStruct(out_shape, F32),
    )(x, W16, b.reshape(1, -1))


def _vq(flat, embeddings, *, B, h):
    M, D = flat.shape
    N = embeddings.shape[0]
    chunk = M // B
    scale = (1.0 + 0.25) / (M * D)
    f2 = jnp.sum(flat ** 2, axis=1, keepdims=True)
    e2 = jnp.sum(embeddings ** 2, axis=1).reshape(1, N)
    q, loss = pl.pallas_call(
        functools.partial(_vq_body, n_embed=N, scale=scale, h=h),
        grid=(B,),
        in_specs=[
            pl.BlockSpec((chunk, D), lambda i: (i, 0)),
            pl.BlockSpec((chunk, 1), lambda i: (i, 0)),
            pl.BlockSpec((N, D), lambda i: (0, 0)),
            pl.BlockSpec((1, N), lambda i: (0, 0)),
        ],
        out_specs=[
            pl.BlockSpec((1, h + 2, h + 2, D), lambda i: (i, 0, 0, 0)),
            pl.BlockSpec((1, 1), lambda i: (0, 0)),
        ],
        out_shape=[
            jax.ShapeDtypeStruct((B, h + 2, h + 2, D), F32),
            jax.ShapeDtypeStruct((1, 1), F32),
        ],
    )(flat, f2, embeddings, e2)
    return q, loss[0, 0]


def _deconv(x, w, b, *, act):
    B, hin2, _, Cin = x.shape
    h = hin2 - 2
    O = w.shape[1]
    V = _deconv_tap_weights(w)
    return pl.pallas_call(
        functools.partial(_deconv_body, h=h, act=act),
        grid=(B, 2, 2),
        in_specs=[
            pl.BlockSpec((1, hin2, hin2, Cin), lambda i, py, px: (i, 0, 0, 0)),
            pl.BlockSpec((1, 1, 4, Cin, O),
                         lambda i, py, px: (py, px, 0, 0, 0)),
            pl.BlockSpec((1, O), lambda i, py, px: (0, 0)),
        ],
        out_specs=pl.BlockSpec((1, 2 * h + 2, 2 * h + 2, O),
                               lambda i, py, px: (i, 0, 0, 0)),
        out_shape=jax.ShapeDtypeStruct((B, 2 * h + 2, 2 * h + 2, O), F32),
    )(x, V, b.reshape(1, -1))


def _deconv3(x, w, b):
    B, hin2, _, Cin = x.shape
    h = hin2 - 2
    V9 = _deconv3_stacked_weights(w)
    b12 = jnp.tile(b.reshape(1, 3), (1, 4))
    return pl.pallas_call(
        functools.partial(_deconv3_body, h=h),
        grid=(B,),
        in_specs=[
            pl.BlockSpec((1, hin2, hin2, Cin), lambda i: (i, 0, 0, 0)),
            pl.BlockSpec(V9.shape, lambda i: (0, 0, 0)),
            pl.BlockSpec((1, 12), lambda i: (0, 0)),
        ],
        out_specs=pl.BlockSpec((1, h * h, 12), lambda i: (i, 0, 0)),
        out_shape=jax.ShapeDtypeStruct((B, h * h, 12), F32),
    )(x, V9, b12)


def kernel(x, conv1_w, conv1_b, conv2_w, conv2_b, conv3_w, conv3_b,
           embeddings, deconv1_w, deconv1_b, deconv2_w, deconv2_b,
           deconv3_w, deconv3_b):
    B = x.shape[0]
    h1 = x.shape[2] // 2
    P1 = _phase_split(jnp.transpose(x, (0, 2, 3, 1)))
    z = _conv1(P1, _conv1_tap_weights(conv1_w), conv1_b,
               B=B, h=h1, Cout=conv1_w.shape[0])
    z = _conv(z, conv2_w, conv2_b, act="relu", pad_out=True)
    z = _conv(z, conv3_w, conv3_b, act="none", pad_out=False)
    h3 = h1 // 4
    D = embeddings.shape[1]
    flat = z.reshape(B * h3 * h3, D)
    q, vq_loss = _vq(flat, embeddings, B=B, h=h3)
    y = _deconv(q, deconv1_w, deconv1_b, act="relu")
    y = _deconv(y, deconv2_w, deconv2_b, act="relu")
    y = _deconv3(y, deconv3_w, deconv3_b)
    h = 2 * h1 // 2
    y = y.reshape(B, h, h, 2, 2, 3)
    y = jnp.transpose(y, (0, 5, 1, 3, 2, 4))
    x_rec = y.reshape(B, 3, 2 * h, 2 * h)
    return (x_rec, vq_loss.reshape(()))
